# baseline (speedup 1.0000x reference)
import functools

import jax
import jax.numpy as jnp
from jax import lax
from jax.experimental import pallas as pl
from jax.experimental.pallas import tpu as pltpu
from jax.experimental.pallas import tpu_sc as plsc


def _transpose_table(emb_t, n_main, d):
    blk = 8192
    grid = (n_main + blk - 1) // blk
    half = blk // 2

    def body(x_ref, o_ref):
        xt = x_ref[...].T
        o_ref[...] = jnp.concatenate([xt[:half], xt[half:]], axis=1)

    out = pl.pallas_call(
        body,
        grid=(grid,),
        in_specs=[pl.BlockSpec((d, blk), lambda k: (0, k))],
        out_specs=pl.BlockSpec((half, 2 * d), lambda k: (k, 0)),
        out_shape=jax.ShapeDtypeStruct((grid * half, 2 * d), jnp.float32),
    )(emb_t)
    return out.reshape(grid * blk, d)


def _transpose_out(out2d, b, h, d):
    bb = 128
    in2 = out2d.reshape(b * h // 2, 2 * d)

    def body(x_ref, o_ref):
        x3 = x_ref[...].reshape(bb, h // 2, 2 * d)
        o_ref[...] = jnp.transpose(x3, (1, 2, 0)).reshape((h // 2) * 2 * d, bb)

    y = pl.pallas_call(
        body,
        grid=(b // bb,),
        in_specs=[pl.BlockSpec((bb * h // 2, 2 * d), lambda k: (k, 0))],
        out_specs=pl.BlockSpec((h * d, bb), lambda k: (0, k)),
        out_shape=jax.ShapeDtypeStruct((h * d, b), jnp.float32),
    )(in2)
    return jnp.transpose(y.reshape(h, d, b), (2, 0, 1))


def _make_gather(n_main, n_new, d, batch):
    info = plsc.get_sparse_core_info()
    nc, ns, nl = info.num_cores, info.num_subcores, info.num_lanes
    nw = nc * ns
    assert batch % nw == 0
    b_per_w = batch // nw
    TBLK = 8192
    TSHIFT = 12
    GROUP = 128
    assert b_per_w % GROUP == 0
    n_groups = b_per_w // GROUP
    NBUF = 8
    assert n_groups % NBUF == 0
    sub_per_group = GROUP // nl

    mesh = plsc.VectorSubcoreMesh(core_axis_name="c", subcore_axis_name="s")

    @functools.partial(
        pl.kernel,
        mesh=mesh,
        out_type=jax.ShapeDtypeStruct((batch, d), jnp.float32),
        compiler_params=pltpu.CompilerParams(
            use_tc_tiling_on_sc=False, needs_layout_passes=False
        ),
        scratch_types=[
            pltpu.VMEM((b_per_w,), jnp.int32),
            pltpu.VMEM((b_per_w,), jnp.int32),
            pltpu.VMEM((n_groups * nl,), jnp.int32),
            pltpu.VMEM((NBUF, GROUP, d), jnp.float32),
            pltpu.VMEM((n_new * d,), jnp.float32),
        ]
        + [pltpu.SemaphoreType.DMA] * (2 * NBUF),
    )
    def gather_kernel(emb_hbm, new_hbm, idx_hbm, out_hbm,
                      idx_v, midx_v, gmax_v, rows_v, new_v, *sems):
        gsems, ssems = sems[:NBUF], sems[NBUF:]
        wid = lax.axis_index("s") * nc + lax.axis_index("c")
        base = wid * b_per_w
        pltpu.sync_copy(new_hbm, new_v)
        pltpu.sync_copy(idx_hbm.at[pl.ds(base, b_per_w)], idx_v)

        def block_body(gb, _):
            copies = []
            for b in range(NBUF):
                g = gb * NBUF + b
                goff = g * GROUP

                @pl.when(gb > 0)
                def _():
                    pltpu.make_async_copy(
                        rows_v.at[b], out_hbm.at[pl.ds(0, GROUP)], ssems[b]
                    ).wait()

                gmax = None
                for s in range(sub_per_group):
                    v = idx_v[pl.ds(goff + s * nl, nl)]
                    mi = jnp.minimum(v, n_main - 1)
                    midx_v[pl.ds(goff + s * nl, nl)] = (
                        (mi & ~(TBLK - 1))
                        + ((mi & (TBLK // 2 - 1)) << 1)
                        + ((mi >> TSHIFT) & 1)
                    )
                    gmax = v if gmax is None else jnp.maximum(gmax, v)
                gmax_v[pl.ds(g * nl, nl)] = gmax
                copies.append(
                    pltpu.async_copy(
                        emb_hbm.at[midx_v.at[pl.ds(goff, GROUP)]],
                        rows_v.at[b],
                        gsems[b],
                    )
                )

            for b in range(NBUF):
                g = gb * NBUF + b
                goff = g * GROUP
                copies[b].wait()
                gmax_s = jnp.max(gmax_v[pl.ds(g * nl, nl)])

                @pl.when(gmax_s >= n_main)
                def _():
                    def fix_body(s, _):
                        off = goff + s * nl
                        smax = jnp.max(idx_v[pl.ds(off, nl)])

                        @pl.when(smax >= n_main)
                        def _():
                            v = idx_v[pl.ds(off, nl)]
                            m = v >= n_main
                            nidx = jnp.clip(v - n_main, 0, n_new - 1)
                            lane = jnp.arange(nl, dtype=jnp.int32)
                            bvec = jnp.full((nl,), b, jnp.int32)
                            rows_ids = s * nl + lane

                            def feat_body(f, _):
                                colf = jnp.full((nl,), 0, jnp.int32) + f
                                vals = plsc.load_gather(
                                    new_v, [nidx * d + colf]
                                )
                                plsc.store_scatter(
                                    rows_v,
                                    [bvec, rows_ids, colf],
                                    vals,
                                    mask=m,
                                )
                                return 0

                            lax.fori_loop(0, d, feat_body, 0)

                        return 0

                    lax.fori_loop(0, sub_per_group, fix_body, 0)

                pltpu.async_copy(
                    rows_v.at[b],
                    out_hbm.at[pl.ds(base + goff, GROUP)],
                    ssems[b],
                )
            return 0

        lax.fori_loop(0, n_groups // NBUF, block_body, 0)
        for b in range(NBUF):
            pltpu.make_async_copy(
                rows_v.at[b], out_hbm.at[pl.ds(0, GROUP)], ssems[b]
            ).wait()

    return gather_kernel


def kernel(x, embedding, new_embedding):
    n_main, d = embedding.shape
    n_new = new_embedding.shape[0]
    b, h = x.shape
    batch = b * h
    idx = x.reshape(-1).astype(jnp.int32)
    table = _transpose_table(embedding.T, n_main, d)
    gather = _make_gather(n_main, n_new, d, batch)
    out = gather(table, new_embedding.reshape(-1), idx)
    return _transpose_out(out, b, h, d)


# speedup vs baseline: 1.9711x; 1.9711x over previous
import functools

import jax
import jax.numpy as jnp
from jax import lax
from jax.experimental import pallas as pl
from jax.experimental.pallas import tpu as pltpu
from jax.experimental.pallas import tpu_sc as plsc


def _transpose_table(emb_t, n_main, d):
    blk = 8192
    grid = (n_main + blk - 1) // blk
    half = blk // 2

    def body(x_ref, o_ref):
        xt = x_ref[...].T
        o_ref[...] = jnp.concatenate([xt[:half], xt[half:]], axis=1)

    out = pl.pallas_call(
        body,
        grid=(grid,),
        in_specs=[pl.BlockSpec((d, blk), lambda k: (0, k))],
        out_specs=pl.BlockSpec((half, 2 * d), lambda k: (k, 0)),
        out_shape=jax.ShapeDtypeStruct((grid * half, 2 * d), jnp.float32),
    )(emb_t)
    return out.reshape(grid * blk, d)


def _transpose_out(out2d, b, h, d):
    bb = 128
    in2 = out2d.reshape(b * h // 2, 2 * d)

    def body(x_ref, o_ref):
        x3 = x_ref[...].reshape(bb, h // 2, 2 * d)
        x4 = jnp.transpose(x3, (1, 0, 2))
        o_ref[...] = jnp.transpose(x4, (0, 2, 1)).reshape((h // 2) * 2 * d, bb)

    y = pl.pallas_call(
        body,
        grid=(b // bb,),
        in_specs=[pl.BlockSpec((bb * h // 2, 2 * d), lambda k: (k, 0))],
        out_specs=pl.BlockSpec((h * d, bb), lambda k: (0, k)),
        out_shape=jax.ShapeDtypeStruct((h * d, b), jnp.float32),
    )(in2)
    return jnp.transpose(y.reshape(h, d, b), (2, 0, 1))


def _make_gather(n_main, n_new, d, batch):
    info = plsc.get_sparse_core_info()
    nc, ns, nl = info.num_cores, info.num_subcores, info.num_lanes
    nw = nc * ns
    assert batch % nw == 0
    b_per_w = batch // nw
    TBLK = 8192
    TSHIFT = 12
    GROUP = 128
    assert b_per_w % GROUP == 0
    n_groups = b_per_w // GROUP
    NBUF = 8
    assert n_groups % NBUF == 0
    sub_per_group = GROUP // nl

    mesh = plsc.VectorSubcoreMesh(core_axis_name="c", subcore_axis_name="s")

    @functools.partial(
        pl.kernel,
        mesh=mesh,
        out_type=jax.ShapeDtypeStruct((batch, d), jnp.float32),
        compiler_params=pltpu.CompilerParams(
            use_tc_tiling_on_sc=False, needs_layout_passes=False
        ),
        scratch_types=[
            pltpu.VMEM((b_per_w,), jnp.int32),
            pltpu.VMEM((b_per_w,), jnp.int32),
            pltpu.VMEM((n_groups * nl,), jnp.int32),
            pltpu.VMEM((NBUF, GROUP, d), jnp.float32),
            pltpu.VMEM((n_new * d,), jnp.float32),
        ]
        + [pltpu.SemaphoreType.DMA] * (2 * NBUF),
    )
    def gather_kernel(emb_hbm, new_hbm, idx_hbm, out_hbm,
                      idx_v, midx_v, gmax_v, rows_v, new_v, *sems):
        gsems, ssems = sems[:NBUF], sems[NBUF:]
        wid = lax.axis_index("s") * nc + lax.axis_index("c")
        base = wid * b_per_w
        pltpu.sync_copy(new_hbm, new_v)
        pltpu.sync_copy(idx_hbm.at[pl.ds(base, b_per_w)], idx_v)

        def block_body(gb, _):
            copies = []
            for b in range(NBUF):
                g = gb * NBUF + b
                goff = g * GROUP

                @pl.when(gb > 0)
                def _():
                    pltpu.make_async_copy(
                        rows_v.at[b], out_hbm.at[pl.ds(0, GROUP)], ssems[b]
                    ).wait()

                gmax = None
                for s in range(sub_per_group):
                    v = idx_v[pl.ds(goff + s * nl, nl)]
                    mi = jnp.minimum(v, n_main - 1)
                    midx_v[pl.ds(goff + s * nl, nl)] = (
                        (mi & ~(TBLK - 1))
                        + ((mi & (TBLK // 2 - 1)) << 1)
                        + ((mi >> TSHIFT) & 1)
                    )
                    gmax = v if gmax is None else jnp.maximum(gmax, v)
                gmax_v[pl.ds(g * nl, nl)] = gmax
                copies.append(
                    pltpu.async_copy(
                        emb_hbm.at[midx_v.at[pl.ds(goff, GROUP)]],
                        rows_v.at[b],
                        gsems[b],
                    )
                )

            for b in range(NBUF):
                g = gb * NBUF + b
                goff = g * GROUP
                copies[b].wait()
                gmax_s = jnp.max(gmax_v[pl.ds(g * nl, nl)])

                @pl.when(gmax_s >= n_main)
                def _():
                    def fix_body(s, _):
                        off = goff + s * nl
                        smax = jnp.max(idx_v[pl.ds(off, nl)])

                        @pl.when(smax >= n_main)
                        def _():
                            v = idx_v[pl.ds(off, nl)]
                            m = v >= n_main
                            nidx = jnp.clip(v - n_main, 0, n_new - 1)
                            lane = jnp.arange(nl, dtype=jnp.int32)
                            bvec = jnp.full((nl,), b, jnp.int32)
                            rows_ids = s * nl + lane

                            def feat_body(f, _):
                                colf = jnp.full((nl,), 0, jnp.int32) + f
                                vals = plsc.load_gather(
                                    new_v, [nidx * d + colf]
                                )
                                plsc.store_scatter(
                                    rows_v,
                                    [bvec, rows_ids, colf],
                                    vals,
                                    mask=m,
                                )
                                return 0

                            lax.fori_loop(0, d, feat_body, 0)

                        return 0

                    lax.fori_loop(0, sub_per_group, fix_body, 0)

                pltpu.async_copy(
                    rows_v.at[b],
                    out_hbm.at[pl.ds(base + goff, GROUP)],
                    ssems[b],
                )
            return 0

        lax.fori_loop(0, n_groups // NBUF, block_body, 0)
        for b in range(NBUF):
            pltpu.make_async_copy(
                rows_v.at[b], out_hbm.at[pl.ds(0, GROUP)], ssems[b]
            ).wait()

    return gather_kernel


def kernel(x, embedding, new_embedding):
    n_main, d = embedding.shape
    n_new = new_embedding.shape[0]
    b, h = x.shape
    batch = b * h
    idx = x.reshape(-1).astype(jnp.int32)
    table = _transpose_table(embedding.T, n_main, d)
    gather = _make_gather(n_main, n_new, d, batch)
    out = gather(table, new_embedding.reshape(-1), idx)
    return _transpose_out(out, b, h, d)


# speedup vs baseline: 2.1101x; 1.0705x over previous
import functools

import jax
import jax.numpy as jnp
from jax import lax
from jax.experimental import pallas as pl
from jax.experimental.pallas import tpu as pltpu
from jax.experimental.pallas import tpu_sc as plsc


_TBLK = 16384
_TSHIFT = (_TBLK // 2).bit_length() - 1


def _transpose_table(emb_t, n_main, d):
    blk = _TBLK
    grid = (n_main + blk - 1) // blk
    half = blk // 2

    def body(x_ref, o_ref):
        xt = x_ref[...].T
        o_ref[...] = jnp.concatenate([xt[:half], xt[half:]], axis=1)

    out = pl.pallas_call(
        body,
        grid=(grid,),
        in_specs=[pl.BlockSpec((d, blk), lambda k: (0, k))],
        out_specs=pl.BlockSpec((half, 2 * d), lambda k: (k, 0)),
        out_shape=jax.ShapeDtypeStruct((grid * half, 2 * d), jnp.float32),
    )(emb_t)
    return out.reshape(grid * blk, d)


def _transpose_out(out2d, b, h, d):
    bb = 128
    in2 = out2d.reshape(b * h // 2, 2 * d)

    def body(x_ref, o_ref):
        x3 = x_ref[...].reshape(bb, h // 2, 2 * d)
        x4 = jnp.transpose(x3, (1, 0, 2))
        o_ref[...] = jnp.transpose(x4, (0, 2, 1)).reshape((h // 2) * 2 * d, bb)

    y = pl.pallas_call(
        body,
        grid=(b // bb,),
        in_specs=[pl.BlockSpec((bb * h // 2, 2 * d), lambda k: (k, 0))],
        out_specs=pl.BlockSpec((h * d, bb), lambda k: (0, k)),
        out_shape=jax.ShapeDtypeStruct((h * d, b), jnp.float32),
    )(in2)
    return jnp.transpose(y.reshape(h, d, b), (2, 0, 1))


def _make_gather(n_main, n_new, d, batch):
    info = plsc.get_sparse_core_info()
    nc, ns, nl = info.num_cores, info.num_subcores, info.num_lanes
    nw = nc * ns
    assert batch % nw == 0
    b_per_w = batch // nw
    TBLK = _TBLK
    TSHIFT = _TSHIFT
    GROUP = 128
    assert b_per_w % GROUP == 0
    n_groups = b_per_w // GROUP
    NBUF = 8
    assert n_groups % NBUF == 0
    sub_per_group = GROUP // nl

    mesh = plsc.VectorSubcoreMesh(core_axis_name="c", subcore_axis_name="s")

    @functools.partial(
        pl.kernel,
        mesh=mesh,
        out_type=jax.ShapeDtypeStruct((batch, d), jnp.float32),
        compiler_params=pltpu.CompilerParams(
            use_tc_tiling_on_sc=False, needs_layout_passes=False
        ),
        scratch_types=[
            pltpu.VMEM((b_per_w,), jnp.int32),
            pltpu.VMEM((b_per_w,), jnp.int32),
            pltpu.VMEM((n_groups * nl,), jnp.int32),
            pltpu.VMEM((NBUF, GROUP, d), jnp.float32),
            pltpu.VMEM((n_new * d,), jnp.float32),
        ]
        + [pltpu.SemaphoreType.DMA] * (2 * NBUF),
    )
    def gather_kernel(emb_hbm, new_hbm, idx_hbm, out_hbm,
                      idx_v, midx_v, gmax_v, rows_v, new_v, *sems):
        gsems, ssems = sems[:NBUF], sems[NBUF:]
        wid = lax.axis_index("s") * nc + lax.axis_index("c")
        base = wid * b_per_w
        pltpu.sync_copy(new_hbm, new_v)
        pltpu.sync_copy(idx_hbm.at[pl.ds(base, b_per_w)], idx_v)

        def block_body(gb, _):
            copies = []
            for b in range(NBUF):
                g = gb * NBUF + b
                goff = g * GROUP

                @pl.when(gb > 0)
                def _():
                    pltpu.make_async_copy(
                        rows_v.at[b], out_hbm.at[pl.ds(0, GROUP)], ssems[b]
                    ).wait()

                gmax = None
                for s in range(sub_per_group):
                    v = idx_v[pl.ds(goff + s * nl, nl)]
                    mi = jnp.minimum(v, n_main - 1)
                    midx_v[pl.ds(goff + s * nl, nl)] = (
                        (mi & ~(TBLK - 1))
                        + ((mi & (TBLK // 2 - 1)) << 1)
                        + ((mi >> TSHIFT) & 1)
                    )
                    gmax = v if gmax is None else jnp.maximum(gmax, v)
                gmax_v[pl.ds(g * nl, nl)] = gmax
                copies.append(
                    pltpu.async_copy(
                        emb_hbm.at[midx_v.at[pl.ds(goff, GROUP)]],
                        rows_v.at[b],
                        gsems[b],
                    )
                )

            for b in range(NBUF):
                g = gb * NBUF + b
                goff = g * GROUP
                copies[b].wait()
                gmax_s = jnp.max(gmax_v[pl.ds(g * nl, nl)])

                @pl.when(gmax_s >= n_main)
                def _():
                    def fix_body(s, _):
                        off = goff + s * nl
                        smax = jnp.max(idx_v[pl.ds(off, nl)])

                        @pl.when(smax >= n_main)
                        def _():
                            v = idx_v[pl.ds(off, nl)]
                            m = v >= n_main
                            nidx = jnp.clip(v - n_main, 0, n_new - 1)
                            lane = jnp.arange(nl, dtype=jnp.int32)
                            bvec = jnp.full((nl,), b, jnp.int32)
                            rows_ids = s * nl + lane

                            def feat_body(f, _):
                                colf = jnp.full((nl,), 0, jnp.int32) + f
                                vals = plsc.load_gather(
                                    new_v, [nidx * d + colf]
                                )
                                plsc.store_scatter(
                                    rows_v,
                                    [bvec, rows_ids, colf],
                                    vals,
                                    mask=m,
                                )
                                return 0

                            lax.fori_loop(0, d, feat_body, 0)

                        return 0

                    lax.fori_loop(0, sub_per_group, fix_body, 0)

                pltpu.async_copy(
                    rows_v.at[b],
                    out_hbm.at[pl.ds(base + goff, GROUP)],
                    ssems[b],
                )
            return 0

        lax.fori_loop(0, n_groups // NBUF, block_body, 0)
        for b in range(NBUF):
            pltpu.make_async_copy(
                rows_v.at[b], out_hbm.at[pl.ds(0, GROUP)], ssems[b]
            ).wait()

    return gather_kernel


def kernel(x, embedding, new_embedding):
    n_main, d = embedding.shape
    n_new = new_embedding.shape[0]
    b, h = x.shape
    batch = b * h
    idx = x.reshape(-1).astype(jnp.int32)
    table = _transpose_table(embedding.T, n_main, d)
    gather = _make_gather(n_main, n_new, d, batch)
    out = gather(table, new_embedding.reshape(-1), idx)
    return _transpose_out(out, b, h, d)


# speedup vs baseline: 2.3547x; 1.1159x over previous
import functools

import jax
import jax.numpy as jnp
from jax import lax
from jax.experimental import pallas as pl
from jax.experimental.pallas import tpu as pltpu
from jax.experimental.pallas import tpu_sc as plsc


_TBLK = 32768
_TSHIFT = (_TBLK // 2).bit_length() - 1


def _transpose_table(emb_t, n_main, d):
    blk = _TBLK
    grid = (n_main + blk - 1) // blk
    half = blk // 2

    def body(x_ref, o_ref):
        xt = x_ref[...].T
        o_ref[...] = jnp.concatenate([xt[:half], xt[half:]], axis=1)

    out = pl.pallas_call(
        body,
        grid=(grid,),
        in_specs=[pl.BlockSpec((d, blk), lambda k: (0, k))],
        out_specs=pl.BlockSpec((half, 2 * d), lambda k: (k, 0)),
        out_shape=jax.ShapeDtypeStruct((grid * half, 2 * d), jnp.float32),
    )(emb_t)
    return out.reshape(grid * blk, d)


def _transpose_out(out2d, b, h, d):
    bb = 256
    in2 = out2d.reshape(b * h // 2, 2 * d)

    def body(x_ref, o_ref):
        x3 = x_ref[...].reshape(bb, h // 2, 2 * d)
        x4 = jnp.transpose(x3, (1, 0, 2))
        o_ref[...] = jnp.transpose(x4, (0, 2, 1)).reshape((h // 2) * 2 * d, bb)

    y = pl.pallas_call(
        body,
        grid=(b // bb,),
        in_specs=[pl.BlockSpec((bb * h // 2, 2 * d), lambda k: (k, 0))],
        out_specs=pl.BlockSpec((h * d, bb), lambda k: (0, k)),
        out_shape=jax.ShapeDtypeStruct((h * d, b), jnp.float32),
    )(in2)
    return jnp.transpose(y.reshape(h, d, b), (2, 0, 1))


def _make_gather(n_main, n_new, d, batch):
    info = plsc.get_sparse_core_info()
    nc, ns, nl = info.num_cores, info.num_subcores, info.num_lanes
    nw = nc * ns
    assert batch % nw == 0
    b_per_w = batch // nw
    TBLK = _TBLK
    TSHIFT = _TSHIFT
    GROUP = 128
    assert b_per_w % GROUP == 0
    n_groups = b_per_w // GROUP
    NBUF = 8
    assert n_groups % NBUF == 0
    sub_per_group = GROUP // nl

    mesh = plsc.VectorSubcoreMesh(core_axis_name="c", subcore_axis_name="s")

    @functools.partial(
        pl.kernel,
        mesh=mesh,
        out_type=jax.ShapeDtypeStruct((batch, d), jnp.float32),
        compiler_params=pltpu.CompilerParams(
            use_tc_tiling_on_sc=False, needs_layout_passes=False
        ),
        scratch_types=[
            pltpu.VMEM((b_per_w,), jnp.int32),
            pltpu.VMEM((b_per_w,), jnp.int32),
            pltpu.VMEM((n_groups * nl,), jnp.int32),
            pltpu.VMEM((NBUF, GROUP, d), jnp.float32),
            pltpu.VMEM((n_new * d,), jnp.float32),
        ]
        + [pltpu.SemaphoreType.DMA] * (2 * NBUF),
    )
    def gather_kernel(emb_hbm, new_hbm, idx_hbm, out_hbm,
                      idx_v, midx_v, gmax_v, rows_v, new_v, *sems):
        gsems, ssems = sems[:NBUF], sems[NBUF:]
        wid = lax.axis_index("s") * nc + lax.axis_index("c")
        base = wid * b_per_w
        pltpu.sync_copy(new_hbm, new_v)
        pltpu.sync_copy(idx_hbm.at[pl.ds(base, b_per_w)], idx_v)

        def block_body(gb, _):
            copies = []
            for b in range(NBUF):
                g = gb * NBUF + b
                goff = g * GROUP

                @pl.when(gb > 0)
                def _():
                    pltpu.make_async_copy(
                        rows_v.at[b], out_hbm.at[pl.ds(0, GROUP)], ssems[b]
                    ).wait()

                gmax = None
                for s in range(sub_per_group):
                    v = idx_v[pl.ds(goff + s * nl, nl)]
                    mi = jnp.minimum(v, n_main - 1)
                    midx_v[pl.ds(goff + s * nl, nl)] = (
                        (mi & ~(TBLK - 1))
                        + ((mi & (TBLK // 2 - 1)) << 1)
                        + ((mi >> TSHIFT) & 1)
                    )
                    gmax = v if gmax is None else jnp.maximum(gmax, v)
                gmax_v[pl.ds(g * nl, nl)] = gmax
                copies.append(
                    pltpu.async_copy(
                        emb_hbm.at[midx_v.at[pl.ds(goff, GROUP)]],
                        rows_v.at[b],
                        gsems[b],
                    )
                )

            for b in range(NBUF):
                g = gb * NBUF + b
                goff = g * GROUP
                copies[b].wait()
                gmax_s = jnp.max(gmax_v[pl.ds(g * nl, nl)])

                @pl.when(gmax_s >= n_main)
                def _():
                    def fix_body(s, _):
                        off = goff + s * nl
                        smax = jnp.max(idx_v[pl.ds(off, nl)])

                        @pl.when(smax >= n_main)
                        def _():
                            v = idx_v[pl.ds(off, nl)]
                            m = v >= n_main
                            nidx = jnp.clip(v - n_main, 0, n_new - 1)
                            lane = jnp.arange(nl, dtype=jnp.int32)
                            bvec = jnp.full((nl,), b, jnp.int32)
                            rows_ids = s * nl + lane

                            def feat_body(f, _):
                                colf = jnp.full((nl,), 0, jnp.int32) + f
                                vals = plsc.load_gather(
                                    new_v, [nidx * d + colf]
                                )
                                plsc.store_scatter(
                                    rows_v,
                                    [bvec, rows_ids, colf],
                                    vals,
                                    mask=m,
                                )
                                return 0

                            lax.fori_loop(0, d, feat_body, 0)

                        return 0

                    lax.fori_loop(0, sub_per_group, fix_body, 0)

                pltpu.async_copy(
                    rows_v.at[b],
                    out_hbm.at[pl.ds(base + goff, GROUP)],
                    ssems[b],
                )
            return 0

        lax.fori_loop(0, n_groups // NBUF, block_body, 0)
        for b in range(NBUF):
            pltpu.make_async_copy(
                rows_v.at[b], out_hbm.at[pl.ds(0, GROUP)], ssems[b]
            ).wait()

    return gather_kernel


def kernel(x, embedding, new_embedding):
    n_main, d = embedding.shape
    n_new = new_embedding.shape[0]
    b, h = x.shape
    batch = b * h
    idx = x.reshape(-1).astype(jnp.int32)
    table = _transpose_table(embedding.T, n_main, d)
    gather = _make_gather(n_main, n_new, d, batch)
    out = gather(table, new_embedding.reshape(-1), idx)
    return _transpose_out(out, b, h, d)


# speedup vs baseline: 2.4768x; 1.0518x over previous
import functools

import jax
import jax.numpy as jnp
from jax import lax
from jax.experimental import pallas as pl
from jax.experimental.pallas import tpu as pltpu
from jax.experimental.pallas import tpu_sc as plsc


_TBLK = 32768
_TSHIFT = (_TBLK // 2).bit_length() - 1


def _transpose_table(emb_t, n_main, d):
    blk = _TBLK
    grid = (n_main + blk - 1) // blk
    half = blk // 2

    def body(x_ref, o_ref):
        xt = x_ref[...].T
        o_ref[...] = jnp.concatenate([xt[:half], xt[half:]], axis=1)

    out = pl.pallas_call(
        body,
        grid=(grid,),
        in_specs=[pl.BlockSpec((d, blk), lambda k: (0, k))],
        out_specs=pl.BlockSpec((half, 2 * d), lambda k: (k, 0)),
        out_shape=jax.ShapeDtypeStruct((grid * half, 2 * d), jnp.float32),
    )(emb_t)
    return out.reshape(grid * blk, d)


def _transpose_out(out2d, b, h, d):
    bb = 512
    in2 = out2d.reshape(b * h // 2, 2 * d)

    def body(x_ref, o_ref):
        x3 = x_ref[...].reshape(bb, h // 2, 2 * d)
        x4 = jnp.transpose(x3, (1, 0, 2))
        o_ref[...] = jnp.transpose(x4, (0, 2, 1)).reshape((h // 2) * 2 * d, bb)

    y = pl.pallas_call(
        body,
        grid=(b // bb,),
        in_specs=[pl.BlockSpec((bb * h // 2, 2 * d), lambda k: (k, 0))],
        out_specs=pl.BlockSpec((h * d, bb), lambda k: (0, k)),
        out_shape=jax.ShapeDtypeStruct((h * d, b), jnp.float32),
    )(in2)
    return jnp.transpose(y.reshape(h, d, b), (2, 0, 1))


def _make_gather(n_main, n_new, d, batch):
    info = plsc.get_sparse_core_info()
    nc, ns, nl = info.num_cores, info.num_subcores, info.num_lanes
    nw = nc * ns
    assert batch % nw == 0
    b_per_w = batch // nw
    TBLK = _TBLK
    TSHIFT = _TSHIFT
    GROUP = 128
    assert b_per_w % GROUP == 0
    n_groups = b_per_w // GROUP
    NBUF = 8
    assert n_groups % NBUF == 0
    sub_per_group = GROUP // nl

    mesh = plsc.VectorSubcoreMesh(core_axis_name="c", subcore_axis_name="s")

    @functools.partial(
        pl.kernel,
        mesh=mesh,
        out_type=jax.ShapeDtypeStruct((batch, d), jnp.float32),
        compiler_params=pltpu.CompilerParams(
            use_tc_tiling_on_sc=False, needs_layout_passes=False
        ),
        scratch_types=[
            pltpu.VMEM((b_per_w,), jnp.int32),
            pltpu.VMEM((b_per_w,), jnp.int32),
            pltpu.VMEM((n_groups * nl,), jnp.int32),
            pltpu.VMEM((NBUF, GROUP, d), jnp.float32),
            pltpu.VMEM((n_new * d,), jnp.float32),
        ]
        + [pltpu.SemaphoreType.DMA] * (2 * NBUF),
    )
    def gather_kernel(emb_hbm, new_hbm, idx_hbm, out_hbm,
                      idx_v, midx_v, gmax_v, rows_v, new_v, *sems):
        gsems, ssems = sems[:NBUF], sems[NBUF:]
        wid = lax.axis_index("s") * nc + lax.axis_index("c")
        base = wid * b_per_w
        pltpu.sync_copy(new_hbm, new_v)
        pltpu.sync_copy(idx_hbm.at[pl.ds(base, b_per_w)], idx_v)

        def block_body(gb, _):
            copies = []
            for b in range(NBUF):
                g = gb * NBUF + b
                goff = g * GROUP

                @pl.when(gb > 0)
                def _():
                    pltpu.make_async_copy(
                        rows_v.at[b], out_hbm.at[pl.ds(0, GROUP)], ssems[b]
                    ).wait()

                gmax = None
                for s in range(sub_per_group):
                    v = idx_v[pl.ds(goff + s * nl, nl)]
                    mi = jnp.minimum(v, n_main - 1)
                    midx_v[pl.ds(goff + s * nl, nl)] = (
                        (mi & ~(TBLK - 1))
                        + ((mi & (TBLK // 2 - 1)) << 1)
                        + ((mi >> TSHIFT) & 1)
                    )
                    gmax = v if gmax is None else jnp.maximum(gmax, v)
                gmax_v[pl.ds(g * nl, nl)] = gmax
                copies.append(
                    pltpu.async_copy(
                        emb_hbm.at[midx_v.at[pl.ds(goff, GROUP)]],
                        rows_v.at[b],
                        gsems[b],
                    )
                )

            for b in range(NBUF):
                g = gb * NBUF + b
                goff = g * GROUP
                copies[b].wait()
                gmax_s = jnp.max(gmax_v[pl.ds(g * nl, nl)])

                @pl.when(gmax_s >= n_main)
                def _():
                    def fix_body(s, _):
                        off = goff + s * nl
                        smax = jnp.max(idx_v[pl.ds(off, nl)])

                        @pl.when(smax >= n_main)
                        def _():
                            v = idx_v[pl.ds(off, nl)]
                            m = v >= n_main
                            nidx = jnp.clip(v - n_main, 0, n_new - 1)
                            lane = jnp.arange(nl, dtype=jnp.int32)
                            bvec = jnp.full((nl,), b, jnp.int32)
                            rows_ids = s * nl + lane

                            def feat_body(f, _):
                                colf = jnp.full((nl,), 0, jnp.int32) + f
                                vals = plsc.load_gather(
                                    new_v, [nidx * d + colf]
                                )
                                plsc.store_scatter(
                                    rows_v,
                                    [bvec, rows_ids, colf],
                                    vals,
                                    mask=m,
                                )
                                return 0

                            lax.fori_loop(0, d, feat_body, 0)

                        return 0

                    lax.fori_loop(0, sub_per_group, fix_body, 0)

                pltpu.async_copy(
                    rows_v.at[b],
                    out_hbm.at[pl.ds(base + goff, GROUP)],
                    ssems[b],
                )
            return 0

        lax.fori_loop(0, n_groups // NBUF, block_body, 0)
        for b in range(NBUF):
            pltpu.make_async_copy(
                rows_v.at[b], out_hbm.at[pl.ds(0, GROUP)], ssems[b]
            ).wait()

    return gather_kernel


def kernel(x, embedding, new_embedding):
    n_main, d = embedding.shape
    n_new = new_embedding.shape[0]
    b, h = x.shape
    batch = b * h
    idx = x.reshape(-1).astype(jnp.int32)
    table = _transpose_table(embedding.T, n_main, d)
    gather = _make_gather(n_main, n_new, d, batch)
    out = gather(table, new_embedding.reshape(-1), idx)
    return _transpose_out(out, b, h, d)


# speedup vs baseline: 2.5279x; 1.0207x over previous
import functools

import jax
import jax.numpy as jnp
from jax import lax
from jax.experimental import pallas as pl
from jax.experimental.pallas import tpu as pltpu
from jax.experimental.pallas import tpu_sc as plsc


_TBLK = 32768
_TSHIFT = (_TBLK // 2).bit_length() - 1


def _transpose_table(emb_t, n_main, d):
    blk = _TBLK
    grid = (n_main + blk - 1) // blk
    half = blk // 2

    def body(x_ref, o_ref):
        xt = x_ref[...].T
        o_ref[...] = jnp.concatenate([xt[:half], xt[half:]], axis=1)

    out = pl.pallas_call(
        body,
        grid=(grid,),
        in_specs=[pl.BlockSpec((d, blk), lambda k: (0, k))],
        out_specs=pl.BlockSpec((half, 2 * d), lambda k: (k, 0)),
        out_shape=jax.ShapeDtypeStruct((grid * half, 2 * d), jnp.float32),
    )(emb_t)
    return out.reshape(grid * blk, d)


def _transpose_out(out2d, b, h, d):
    bb = 1024
    in2 = out2d.reshape(b * h // 2, 2 * d)

    def body(x_ref, o_ref):
        x3 = x_ref[...].reshape(bb, h // 2, 2 * d)
        x4 = jnp.transpose(x3, (1, 0, 2))
        o_ref[...] = jnp.transpose(x4, (0, 2, 1)).reshape((h // 2) * 2 * d, bb)

    y = pl.pallas_call(
        body,
        grid=(b // bb,),
        in_specs=[pl.BlockSpec((bb * h // 2, 2 * d), lambda k: (k, 0))],
        out_specs=pl.BlockSpec((h * d, bb), lambda k: (0, k)),
        out_shape=jax.ShapeDtypeStruct((h * d, b), jnp.float32),
    )(in2)
    return jnp.transpose(y.reshape(h, d, b), (2, 0, 1))


def _make_gather(n_main, n_new, d, batch):
    info = plsc.get_sparse_core_info()
    nc, ns, nl = info.num_cores, info.num_subcores, info.num_lanes
    nw = nc * ns
    assert batch % nw == 0
    b_per_w = batch // nw
    TBLK = _TBLK
    TSHIFT = _TSHIFT
    GROUP = 128
    assert b_per_w % GROUP == 0
    n_groups = b_per_w // GROUP
    NBUF = 8
    assert n_groups % NBUF == 0
    sub_per_group = GROUP // nl

    mesh = plsc.VectorSubcoreMesh(core_axis_name="c", subcore_axis_name="s")

    @functools.partial(
        pl.kernel,
        mesh=mesh,
        out_type=jax.ShapeDtypeStruct((batch, d), jnp.float32),
        compiler_params=pltpu.CompilerParams(
            use_tc_tiling_on_sc=False, needs_layout_passes=False
        ),
        scratch_types=[
            pltpu.VMEM((b_per_w,), jnp.int32),
            pltpu.VMEM((b_per_w,), jnp.int32),
            pltpu.VMEM((n_groups * nl,), jnp.int32),
            pltpu.VMEM((NBUF, GROUP, d), jnp.float32),
            pltpu.VMEM((n_new * d,), jnp.float32),
        ]
        + [pltpu.SemaphoreType.DMA] * (2 * NBUF),
    )
    def gather_kernel(emb_hbm, new_hbm, idx_hbm, out_hbm,
                      idx_v, midx_v, gmax_v, rows_v, new_v, *sems):
        gsems, ssems = sems[:NBUF], sems[NBUF:]
        wid = lax.axis_index("s") * nc + lax.axis_index("c")
        base = wid * b_per_w
        pltpu.sync_copy(new_hbm, new_v)
        pltpu.sync_copy(idx_hbm.at[pl.ds(base, b_per_w)], idx_v)

        def block_body(gb, _):
            copies = []
            for b in range(NBUF):
                g = gb * NBUF + b
                goff = g * GROUP

                @pl.when(gb > 0)
                def _():
                    pltpu.make_async_copy(
                        rows_v.at[b], out_hbm.at[pl.ds(0, GROUP)], ssems[b]
                    ).wait()

                gmax = None
                for s in range(sub_per_group):
                    v = idx_v[pl.ds(goff + s * nl, nl)]
                    mi = jnp.minimum(v, n_main - 1)
                    midx_v[pl.ds(goff + s * nl, nl)] = (
                        (mi & ~(TBLK - 1))
                        + ((mi & (TBLK // 2 - 1)) << 1)
                        + ((mi >> TSHIFT) & 1)
                    )
                    gmax = v if gmax is None else jnp.maximum(gmax, v)
                gmax_v[pl.ds(g * nl, nl)] = gmax
                copies.append(
                    pltpu.async_copy(
                        emb_hbm.at[midx_v.at[pl.ds(goff, GROUP)]],
                        rows_v.at[b],
                        gsems[b],
                    )
                )

            for b in range(NBUF):
                g = gb * NBUF + b
                goff = g * GROUP
                copies[b].wait()
                gmax_s = jnp.max(gmax_v[pl.ds(g * nl, nl)])

                @pl.when(gmax_s >= n_main)
                def _():
                    def fix_body(s, _):
                        off = goff + s * nl
                        smax = jnp.max(idx_v[pl.ds(off, nl)])

                        @pl.when(smax >= n_main)
                        def _():
                            v = idx_v[pl.ds(off, nl)]
                            m = v >= n_main
                            nidx = jnp.clip(v - n_main, 0, n_new - 1)
                            lane = jnp.arange(nl, dtype=jnp.int32)
                            bvec = jnp.full((nl,), b, jnp.int32)
                            rows_ids = s * nl + lane

                            def feat_body(f, _):
                                colf = jnp.full((nl,), 0, jnp.int32) + f
                                vals = plsc.load_gather(
                                    new_v, [nidx * d + colf]
                                )
                                plsc.store_scatter(
                                    rows_v,
                                    [bvec, rows_ids, colf],
                                    vals,
                                    mask=m,
                                )
                                return 0

                            lax.fori_loop(0, d, feat_body, 0)

                        return 0

                    lax.fori_loop(0, sub_per_group, fix_body, 0)

                pltpu.async_copy(
                    rows_v.at[b],
                    out_hbm.at[pl.ds(base + goff, GROUP)],
                    ssems[b],
                )
            return 0

        lax.fori_loop(0, n_groups // NBUF, block_body, 0)
        for b in range(NBUF):
            pltpu.make_async_copy(
                rows_v.at[b], out_hbm.at[pl.ds(0, GROUP)], ssems[b]
            ).wait()

    return gather_kernel


def kernel(x, embedding, new_embedding):
    n_main, d = embedding.shape
    n_new = new_embedding.shape[0]
    b, h = x.shape
    batch = b * h
    idx = x.reshape(-1).astype(jnp.int32)
    table = _transpose_table(embedding.T, n_main, d)
    gather = _make_gather(n_main, n_new, d, batch)
    out = gather(table, new_embedding.reshape(-1), idx)
    return _transpose_out(out, b, h, d)
